# baseline (speedup 1.0000x reference)
import functools

import jax
import jax.numpy as jnp
from jax import lax
from jax.experimental import pallas as pl
from jax.experimental.pallas import tpu as pltpu
from jax.experimental.pallas import tpu_sc as plsc

B, T = 1024, 200
VOCAB = 1024
D = 128
N = B * T
NC, NS = 2, 16
NW = NC * NS
TOK_PER_W = N // NW
C = 128
NCHUNK = TOK_PER_W // C
NSET = 4
ROWS_PER_TILE = 4 * VOCAB // NS

_mesh = plsc.VectorSubcoreMesh(core_axis_name="c", subcore_axis_name="s")


@functools.partial(
    pl.kernel,
    mesh=_mesh,
    out_type=jax.ShapeDtypeStruct((N, D), jnp.float32),
    scratch_types=[
        pltpu.VMEM_SHARED((4 * VOCAB, D), jnp.float32),
        pltpu.VMEM((4, TOK_PER_W), jnp.int32),
        pltpu.VMEM((C, D), jnp.float32),
        pltpu.VMEM((C, D), jnp.float32),
        pltpu.VMEM((C, D), jnp.float32),
        pltpu.VMEM((C, D), jnp.float32),
        pltpu.SemaphoreType.DMA,
        pltpu.SemaphoreType.DMA,
        pltpu.SemaphoreType.DMA,
        pltpu.SemaphoreType.DMA,
        pltpu.SemaphoreType.DMA,
        pltpu.SemaphoreType.DMA,
        pltpu.SemaphoreType.DMA,
        pltpu.SemaphoreType.DMA,
        pltpu.SemaphoreType.DMA,
        pltpu.SemaphoreType.DMA,
        pltpu.SemaphoreType.DMA,
        pltpu.SemaphoreType.DMA,
        pltpu.SemaphoreType.DMA,
        pltpu.SemaphoreType.DMA,
        pltpu.SemaphoreType.DMA,
        pltpu.SemaphoreType.DMA,
    ],
)
def _emb_sum_kernel(w_hbm, idx_hbm, out_hbm,
                    w_sh, idxv, acc0, acc1, acc2, acc3,
                    bs0, bs1, bs2, bs3, as0, as1, as2, as3,
                    os0, os1, os2, os3, hs0, hs1, hs2, hs3):
    cid = lax.axis_index("c")
    sid = lax.axis_index("s")
    wid = sid * NC + cid
    base0 = wid * TOK_PER_W
    accs = (acc0, acc1, acc2, acc3)
    bsems = (bs0, bs1, bs2, bs3)
    asems = (as0, as1, as2, as3)
    osems = (os0, os1, os2, os3)
    hsems = (hs0, hs1, hs2, hs3)
    w_srcs = (w_sh, w_sh, w_hbm, w_hbm)

    pltpu.sync_copy(w_hbm.at[pl.ds(sid * ROWS_PER_TILE, ROWS_PER_TILE)],
                    w_sh.at[pl.ds(sid * ROWS_PER_TILE, ROWS_PER_TILE)])
    for k in range(4):
        pltpu.sync_copy(idx_hbm.at[k, pl.ds(base0, TOK_PER_W)], idxv.at[k])
    plsc.subcore_barrier()

    def base_cp(ci, s):
        return pltpu.make_async_copy(
            w_sh.at[idxv.at[0, pl.ds(ci * C, C)]], accs[s], bsems[s])

    def add_start(ci, s, k):
        sem = hsems[s] if k >= 2 else asems[s]
        pltpu.async_copy(
            w_srcs[k].at[idxv.at[k, pl.ds(ci * C, C)]], accs[s], sem,
            add=True)

    def add_wait(ci, s, k):
        sem = hsems[s] if k >= 2 else asems[s]
        pltpu.make_async_copy(
            w_srcs[k].at[idxv.at[k, pl.ds(ci * C, C)]], accs[s], sem).wait()

    def out_cp(ci, s):
        return pltpu.make_async_copy(
            accs[s], out_hbm.at[pl.ds(base0 + ci * C, C)], osems[s])

    def step(ci, d, do_a, wait_b, issue_b, do_c):
        if do_a:
            sa = (d - 1) % NSET
            for k in range(1, 4):
                add_wait(ci - 1, sa, k)
            out_cp(ci - 1, sa).start()
        sb = (d + 1) % NSET
        if wait_b:
            out_cp(ci - 3, sb).wait()
        if issue_b:
            base_cp(ci + 1, sb).start()
        if do_c:
            sc_ = d % NSET
            base_cp(ci, sc_).wait()
            for k in range(1, 4):
                add_start(ci, sc_, k)

    base_cp(0, 0).start()
    step(0, 0, do_a=False, wait_b=False, issue_b=True, do_c=True)
    step(1, 1, do_a=True, wait_b=False, issue_b=True, do_c=True)
    step(2, 2, do_a=True, wait_b=False, issue_b=True, do_c=True)
    step(3, 3, do_a=True, wait_b=True, issue_b=True, do_c=True)

    def body(g, carry):
        ci = 4 * g
        for d in range(4):
            step(ci + d, d, do_a=True, wait_b=True, issue_b=True, do_c=True)
        return carry

    lax.fori_loop(1, NCHUNK // 4, body, 0)
    ci_t = NCHUNK - 2
    step(ci_t, ci_t % NSET, do_a=True, wait_b=True, issue_b=True, do_c=True)
    step(ci_t + 1, (ci_t + 1) % NSET, do_a=True, wait_b=True, issue_b=False,
         do_c=True)
    s_last = (NCHUNK - 1) % NSET
    for k in range(1, 4):
        add_wait(NCHUNK - 1, s_last, k)
    out_cp(NCHUNK - 1, s_last).start()
    for ci in (NCHUNK - 3, NCHUNK - 2, NCHUNK - 1):
        out_cp(ci, ci % NSET).wait()


def kernel(x, W_sy, W_tone, W_syllable_flag, W_ws):
    w_cat = jnp.concatenate([W_sy, W_tone, W_syllable_flag, W_ws], axis=0)
    idx = x.reshape(N, 4) + jnp.arange(4, dtype=jnp.int32) * VOCAB
    idx_all = idx.T
    out = _emb_sum_kernel(w_cat, idx_all)
    return out.reshape(B, T, D)


# speedup vs baseline: 1.0582x; 1.0582x over previous
import functools

import jax
import jax.numpy as jnp
from jax import lax
from jax.experimental import pallas as pl
from jax.experimental.pallas import tpu as pltpu
from jax.experimental.pallas import tpu_sc as plsc

B, T = 1024, 200
VOCAB = 1024
D = 128
N = B * T
NC, NS = 2, 16
NW = NC * NS
TOK_PER_W = N // NW
C = 128
NCHUNK = TOK_PER_W // C
NSET = 4
ROWS_PER_TILE = 4 * VOCAB // NS

_mesh = plsc.VectorSubcoreMesh(core_axis_name="c", subcore_axis_name="s")


@functools.partial(
    pl.kernel,
    mesh=_mesh,
    out_type=jax.ShapeDtypeStruct((N, D), jnp.float32),
    scratch_types=[
        pltpu.VMEM_SHARED((4 * VOCAB, D), jnp.float32),
        pltpu.VMEM((4, TOK_PER_W), jnp.int32),
        pltpu.VMEM((C, D), jnp.float32),
        pltpu.VMEM((C, D), jnp.float32),
        pltpu.VMEM((C, D), jnp.float32),
        pltpu.VMEM((C, D), jnp.float32),
        pltpu.SemaphoreType.DMA,
        pltpu.SemaphoreType.DMA,
        pltpu.SemaphoreType.DMA,
        pltpu.SemaphoreType.DMA,
        pltpu.SemaphoreType.DMA,
        pltpu.SemaphoreType.DMA,
        pltpu.SemaphoreType.DMA,
        pltpu.SemaphoreType.DMA,
        pltpu.SemaphoreType.DMA,
        pltpu.SemaphoreType.DMA,
        pltpu.SemaphoreType.DMA,
        pltpu.SemaphoreType.DMA,
        pltpu.SemaphoreType.DMA,
        pltpu.SemaphoreType.DMA,
        pltpu.SemaphoreType.DMA,
        pltpu.SemaphoreType.DMA,
    ],
)
def _emb_sum_kernel(w_hbm, idx_hbm, out_hbm,
                    w_sh, idxv, acc0, acc1, acc2, acc3,
                    bs0, bs1, bs2, bs3, as0, as1, as2, as3,
                    os0, os1, os2, os3, hs0, hs1, hs2, hs3):
    cid = lax.axis_index("c")
    sid = lax.axis_index("s")
    wid = sid * NC + cid
    base0 = wid * TOK_PER_W
    accs = (acc0, acc1, acc2, acc3)
    bsems = (bs0, bs1, bs2, bs3)
    asems = (as0, as1, as2, as3)
    osems = (os0, os1, os2, os3)
    hsems = (hs0, hs1, hs2, hs3)
    w_srcs = (w_sh, w_sh, w_sh, w_hbm)

    pltpu.sync_copy(w_hbm.at[pl.ds(sid * ROWS_PER_TILE, ROWS_PER_TILE)],
                    w_sh.at[pl.ds(sid * ROWS_PER_TILE, ROWS_PER_TILE)])
    for k in range(4):
        pltpu.sync_copy(idx_hbm.at[k, pl.ds(base0, TOK_PER_W)], idxv.at[k])
    plsc.subcore_barrier()

    def base_cp(ci, s):
        return pltpu.make_async_copy(
            w_sh.at[idxv.at[0, pl.ds(ci * C, C)]], accs[s], bsems[s])

    def add_start(ci, s, k):
        sem = hsems[s] if k == 3 else asems[s]
        pltpu.async_copy(
            w_srcs[k].at[idxv.at[k, pl.ds(ci * C, C)]], accs[s], sem,
            add=True)

    def add_wait(ci, s, k):
        sem = hsems[s] if k == 3 else asems[s]
        pltpu.make_async_copy(
            w_srcs[k].at[idxv.at[k, pl.ds(ci * C, C)]], accs[s], sem).wait()

    def out_cp(ci, s):
        return pltpu.make_async_copy(
            accs[s], out_hbm.at[pl.ds(base0 + ci * C, C)], osems[s])

    def step(ci, d, do_a, wait_b, issue_b, do_c):
        if do_a:
            sa = (d - 1) % NSET
            for k in range(1, 4):
                add_wait(ci - 1, sa, k)
            out_cp(ci - 1, sa).start()
        sb = (d + 1) % NSET
        if wait_b:
            out_cp(ci - 3, sb).wait()
        if issue_b:
            base_cp(ci + 1, sb).start()
        if do_c:
            sc_ = d % NSET
            base_cp(ci, sc_).wait()
            for k in range(1, 4):
                add_start(ci, sc_, k)

    base_cp(0, 0).start()
    step(0, 0, do_a=False, wait_b=False, issue_b=True, do_c=True)
    step(1, 1, do_a=True, wait_b=False, issue_b=True, do_c=True)
    step(2, 2, do_a=True, wait_b=False, issue_b=True, do_c=True)
    step(3, 3, do_a=True, wait_b=True, issue_b=True, do_c=True)

    def body(g, carry):
        ci = 4 * g
        for d in range(4):
            step(ci + d, d, do_a=True, wait_b=True, issue_b=True, do_c=True)
        return carry

    lax.fori_loop(1, NCHUNK // 4, body, 0)
    ci_t = NCHUNK - 2
    step(ci_t, ci_t % NSET, do_a=True, wait_b=True, issue_b=True, do_c=True)
    step(ci_t + 1, (ci_t + 1) % NSET, do_a=True, wait_b=True, issue_b=False,
         do_c=True)
    s_last = (NCHUNK - 1) % NSET
    for k in range(1, 4):
        add_wait(NCHUNK - 1, s_last, k)
    out_cp(NCHUNK - 1, s_last).start()
    for ci in (NCHUNK - 3, NCHUNK - 2, NCHUNK - 1):
        out_cp(ci, ci % NSET).wait()


def kernel(x, W_sy, W_tone, W_syllable_flag, W_ws):
    w_cat = jnp.concatenate([W_sy, W_tone, W_syllable_flag, W_ws], axis=0)
    idx = x.reshape(N, 4) + jnp.arange(4, dtype=jnp.int32) * VOCAB
    idx_all = idx.T
    out = _emb_sum_kernel(w_cat, idx_all)
    return out.reshape(B, T, D)
